# baseline (speedup 1.0000x reference)
import functools

import jax
import jax.numpy as jnp
from jax import lax
from jax.experimental import pallas as pl
from jax.experimental.pallas import tpu as pltpu
from jax.experimental.pallas import tpu_sc as plsc

_TN = 512
_C = 64
_NC = 2
_NS = 16


def _top3_body(x2p_ref, x1p_ref, n2_ref, n1_ref, gidx_ref, wts_ref):
    x2p = x2p_ref[0]
    x1p = x1p_ref[0]
    S = x2p.shape[0]
    b = pl.program_id(0)

    mm = jax.lax.dot_general(
        x2p, x1p, (((1,), (0,)), ((), ())),
        preferred_element_type=jnp.float32)
    e0 = n2_ref[0] - 2.0 * mm
    iota = jax.lax.broadcasted_iota(jnp.int32, e0.shape, 0)

    e = e0
    ms = []
    for _ in range(3):
        m = jnp.min(e, axis=0, keepdims=True)
        ms.append(m)
        e = jnp.where(e == m, jnp.inf, e)

    rs = []
    for k in range(3):
        idx = jnp.min(jnp.where(e0 == ms[k], iota, S), axis=0,
                      keepdims=True)
        gidx_ref[pl.ds(k, 1), :] = idx + b * S
        rs.append(1.0 / (ms[k] + n1_ref[0] + 1e-8))
    norm = rs[0] + rs[1] + rs[2]
    for k in range(3):
        wts_ref[pl.ds(k, 1), :] = rs[k] / norm


def _bag_body(p2l_hbm, gidx_hbm, wts_hbm, interp_hbm,
              i0_v, i1_v, i2_v, w0_v, w1_v, w2_v,
              r0_v, r1_v, r2_v, out_v, sem):
    C = _C
    D2 = p2l_hbm.shape[1]
    Q = gidx_hbm.shape[0] // 3
    QW = Q // (_NC * _NS)
    wid = lax.axis_index("s") * _NC + lax.axis_index("c")

    def chunk(c, carry):
        qb = wid * QW + c * C
        pltpu.sync_copy(gidx_hbm.at[pl.ds(qb, C)], i0_v)
        pltpu.sync_copy(gidx_hbm.at[pl.ds(Q + qb, C)], i1_v)
        pltpu.sync_copy(gidx_hbm.at[pl.ds(2 * Q + qb, C)], i2_v)
        pltpu.sync_copy(wts_hbm.at[pl.ds(qb, C)], w0_v)
        pltpu.sync_copy(wts_hbm.at[pl.ds(Q + qb, C)], w1_v)
        pltpu.sync_copy(wts_hbm.at[pl.ds(2 * Q + qb, C)], w2_v)
        pltpu.async_copy(p2l_hbm.at[i0_v], r0_v, sem).wait()
        pltpu.async_copy(p2l_hbm.at[i1_v], r1_v, sem).wait()
        pltpu.async_copy(p2l_hbm.at[i2_v], r2_v, sem).wait()

        def group(g, carry2):
            base = g * 16
            w0v = w0_v[pl.ds(base, 16)]
            w1v = w1_v[pl.ds(base, 16)]
            w2v = w2_v[pl.ds(base, 16)]
            for ii in range(16):
                w0, w1, w2 = w0v[ii], w1v[ii], w2v[ii]
                i = base + ii
                for j in range(D2 // 16):
                    sl = pl.ds(j * 16, 16)
                    out_v[i, sl] = (w0 * r0_v[i, sl] + w1 * r1_v[i, sl]
                                    + w2 * r2_v[i, sl])
            return carry2

        lax.fori_loop(0, C // 16, group, 0)
        pltpu.sync_copy(out_v, interp_hbm.at[pl.ds(qb, C)])
        return carry

    lax.fori_loop(0, QW // C, chunk, 0)


def _mlp_body(interp_ref, p1_ref, w1a_ref, w1b_ref, w2_ref, b1_ref, b2_ref,
              out_ref):
    h = (jnp.dot(w1a_ref[...], p1_ref[0],
                 preferred_element_type=jnp.float32)
         + jax.lax.dot_general(
             w1b_ref[...], interp_ref[...], (((1,), (1,)), ((), ())),
             preferred_element_type=jnp.float32)
         + b1_ref[...])
    h = jnp.maximum(h, 0.0)
    o = jnp.dot(w2_ref[...], h,
                preferred_element_type=jnp.float32) + b2_ref[...]
    out_ref[0] = jnp.maximum(o, 0.0)


def kernel(xyz1, xyz2, points1, points2, W1, b1, W2, b2):
    B, _, N = xyz1.shape
    S = xyz2.shape[2]
    D1 = points1.shape[1]
    D2 = points2.shape[1]
    H = W1.shape[0]
    O = W2.shape[0]
    TN = _TN
    NT = N // TN
    Q = B * N

    n1 = jnp.sum(xyz1 * xyz1, axis=1, keepdims=True)
    n2 = jnp.transpose(jnp.sum(xyz2 * xyz2, axis=1, keepdims=True),
                       (0, 2, 1))
    z1 = jnp.zeros((B, 5, N), jnp.float32)
    z2 = jnp.zeros((B, 5, S), jnp.float32)
    x1p = jnp.concatenate([xyz1, z1], axis=1)
    x2p = jnp.transpose(
        jnp.concatenate([xyz2, z2], axis=1), (0, 2, 1))

    gidx, wts = pl.pallas_call(
        _top3_body,
        grid=(B, NT),
        in_specs=[
            pl.BlockSpec((1, S, 8), lambda b, t: (b, 0, 0)),
            pl.BlockSpec((1, 8, TN), lambda b, t: (b, 0, t)),
            pl.BlockSpec((1, S, 1), lambda b, t: (b, 0, 0)),
            pl.BlockSpec((1, 1, TN), lambda b, t: (b, 0, t)),
        ],
        out_specs=[
            pl.BlockSpec((3, TN), lambda b, t: (0, b * (N // _TN) + t)),
            pl.BlockSpec((3, TN), lambda b, t: (0, b * (N // _TN) + t)),
        ],
        out_shape=[
            jax.ShapeDtypeStruct((3, Q), jnp.int32),
            jax.ShapeDtypeStruct((3, Q), jnp.float32),
        ],
        compiler_params=pltpu.CompilerParams(
            dimension_semantics=("parallel", "parallel")),
    )(x2p, x1p, n2, n1)

    p2l = jnp.reshape(jnp.transpose(points2, (0, 2, 1)), (B * S, D2))
    mesh = plsc.VectorSubcoreMesh(core_axis_name="c", subcore_axis_name="s")
    bag = functools.partial(
        pl.kernel,
        mesh=mesh,
        out_type=jax.ShapeDtypeStruct((Q, D2), jnp.float32),
        scratch_types=[
            pltpu.VMEM((_C,), jnp.int32),
            pltpu.VMEM((_C,), jnp.int32),
            pltpu.VMEM((_C,), jnp.int32),
            pltpu.VMEM((_C,), jnp.float32),
            pltpu.VMEM((_C,), jnp.float32),
            pltpu.VMEM((_C,), jnp.float32),
            pltpu.VMEM((_C, D2), jnp.float32),
            pltpu.VMEM((_C, D2), jnp.float32),
            pltpu.VMEM((_C, D2), jnp.float32),
            pltpu.VMEM((_C, D2), jnp.float32),
            pltpu.SemaphoreType.DMA,
        ],
    )(_bag_body)
    interp = bag(p2l, jnp.reshape(gidx, (3 * Q,)),
                 jnp.reshape(wts, (3 * Q,)))

    W1a = W1[:, :D1]
    W1b = W1[:, D1:]
    b1c = b1[:, None]
    b2c = b2[:, None]
    out = pl.pallas_call(
        _mlp_body,
        grid=(B, NT),
        in_specs=[
            pl.BlockSpec((TN, D2), lambda b, t: (b * (N // _TN) + t, 0)),
            pl.BlockSpec((1, D1, TN), lambda b, t: (b, 0, t)),
            pl.BlockSpec((H, D1), lambda b, t: (0, 0)),
            pl.BlockSpec((H, D2), lambda b, t: (0, 0)),
            pl.BlockSpec((O, H), lambda b, t: (0, 0)),
            pl.BlockSpec((H, 1), lambda b, t: (0, 0)),
            pl.BlockSpec((O, 1), lambda b, t: (0, 0)),
        ],
        out_specs=pl.BlockSpec((1, O, TN), lambda b, t: (b, 0, t)),
        out_shape=jax.ShapeDtypeStruct((B, O, N), jnp.float32),
        compiler_params=pltpu.CompilerParams(
            dimension_semantics=("parallel", "parallel")),
    )(interp, points1, W1a, W1b, W2, b1c, b2c)
    return out


# speedup vs baseline: 1.1057x; 1.1057x over previous
import functools

import jax
import jax.numpy as jnp
from jax import lax
from jax.experimental import pallas as pl
from jax.experimental.pallas import tpu as pltpu
from jax.experimental.pallas import tpu_sc as plsc

_TN = 512
_C = 32
_NC = 2
_NS = 16


def _top3_body(x2p_ref, x1p_ref, n2_ref, n1_ref, gidx_ref, wts_ref):
    x2p = x2p_ref[0]
    x1p = x1p_ref[0]
    S = x2p.shape[0]
    b = pl.program_id(0)

    mm = jax.lax.dot_general(
        x2p, x1p, (((1,), (0,)), ((), ())),
        preferred_element_type=jnp.float32)
    e0 = n2_ref[0] - 2.0 * mm
    iota = jax.lax.broadcasted_iota(jnp.int32, e0.shape, 0)

    e = e0
    ms = []
    for _ in range(3):
        m = jnp.min(e, axis=0, keepdims=True)
        ms.append(m)
        e = jnp.where(e == m, jnp.inf, e)

    rs = []
    for k in range(3):
        idx = jnp.min(jnp.where(e0 == ms[k], iota, S), axis=0,
                      keepdims=True)
        gidx_ref[pl.ds(k, 1), :] = idx + b * S
        rs.append(1.0 / (ms[k] + n1_ref[0] + 1e-8))
    norm = rs[0] + rs[1] + rs[2]
    for k in range(3):
        wts_ref[pl.ds(k, 1), :] = rs[k] / norm


def _bag_body(p2l_hbm, gidx_hbm, wts_hbm, interp_hbm,
              i0_v, i1_v, i2_v, w0_v, w1_v, w2_v,
              ra0, ra1, ra2, rb0, rb1, rb2, out_v, sema, semb):
    C = _C
    D2 = p2l_hbm.shape[1]
    Q = gidx_hbm.shape[0] // 3
    QW = Q // (_NC * _NS)
    NCH = QW // C
    wid = lax.axis_index("s") * _NC + lax.axis_index("c")
    span = wid * QW

    pltpu.sync_copy(gidx_hbm.at[pl.ds(span, QW)], i0_v)
    pltpu.sync_copy(gidx_hbm.at[pl.ds(Q + span, QW)], i1_v)
    pltpu.sync_copy(gidx_hbm.at[pl.ds(2 * Q + span, QW)], i2_v)
    pltpu.sync_copy(wts_hbm.at[pl.ds(span, QW)], w0_v)
    pltpu.sync_copy(wts_hbm.at[pl.ds(Q + span, QW)], w1_v)
    pltpu.sync_copy(wts_hbm.at[pl.ds(2 * Q + span, QW)], w2_v)

    def issue(c, bufs, sem):
        off = c * C
        r0, r1, r2 = bufs
        pltpu.async_copy(p2l_hbm.at[i0_v.at[pl.ds(off, C)]], r0, sem)
        pltpu.async_copy(p2l_hbm.at[i1_v.at[pl.ds(off, C)]], r1, sem)
        pltpu.async_copy(p2l_hbm.at[i2_v.at[pl.ds(off, C)]], r2, sem)

    def drain(c, bufs, sem):
        off = c * C
        r0, r1, r2 = bufs
        pltpu.make_async_copy(p2l_hbm.at[i0_v.at[pl.ds(off, C)]], r0, sem).wait()
        pltpu.make_async_copy(p2l_hbm.at[i1_v.at[pl.ds(off, C)]], r1, sem).wait()
        pltpu.make_async_copy(p2l_hbm.at[i2_v.at[pl.ds(off, C)]], r2, sem).wait()

    def compute(c, bufs):
        r0_v, r1_v, r2_v = bufs

        def group(g, carry2):
            base = g * 16
            w0v = w0_v[pl.ds(c * C + base, 16)]
            w1v = w1_v[pl.ds(c * C + base, 16)]
            w2v = w2_v[pl.ds(c * C + base, 16)]
            for ii in range(16):
                w0, w1, w2 = w0v[ii], w1v[ii], w2v[ii]
                i = base + ii
                for j in range(D2 // 16):
                    sl = pl.ds(j * 16, 16)
                    out_v[i, sl] = (w0 * r0_v[i, sl] + w1 * r1_v[i, sl]
                                    + w2 * r2_v[i, sl])
            return carry2

        lax.fori_loop(0, C // 16, group, 0)
        pltpu.sync_copy(out_v, interp_hbm.at[pl.ds(span + c * C, C)])

    bufs_a = (ra0, ra1, ra2)
    bufs_b = (rb0, rb1, rb2)
    issue(0, bufs_a, sema)

    def pair(p, carry):
        c0 = 2 * p
        c1 = c0 + 1
        issue(c1, bufs_b, semb)
        drain(c0, bufs_a, sema)
        compute(c0, bufs_a)

        @pl.when(c0 + 2 < NCH)
        def _():
            issue(c0 + 2, bufs_a, sema)

        drain(c1, bufs_b, semb)
        compute(c1, bufs_b)
        return carry

    lax.fori_loop(0, NCH // 2, pair, 0)


def _mlp_body(interp_ref, p1_ref, w1a_ref, w1b_ref, w2_ref, b1_ref, b2_ref,
              out_ref):
    h = (jnp.dot(w1a_ref[...], p1_ref[0],
                 preferred_element_type=jnp.float32)
         + jax.lax.dot_general(
             w1b_ref[...], interp_ref[...], (((1,), (1,)), ((), ())),
             preferred_element_type=jnp.float32)
         + b1_ref[...])
    h = jnp.maximum(h, 0.0)
    o = jnp.dot(w2_ref[...], h,
                preferred_element_type=jnp.float32) + b2_ref[...]
    out_ref[0] = jnp.maximum(o, 0.0)


def kernel(xyz1, xyz2, points1, points2, W1, b1, W2, b2):
    B, _, N = xyz1.shape
    S = xyz2.shape[2]
    D1 = points1.shape[1]
    D2 = points2.shape[1]
    H = W1.shape[0]
    O = W2.shape[0]
    TN = _TN
    NT = N // TN
    Q = B * N

    n1 = jnp.sum(xyz1 * xyz1, axis=1, keepdims=True)
    n2 = jnp.transpose(jnp.sum(xyz2 * xyz2, axis=1, keepdims=True),
                       (0, 2, 1))
    z1 = jnp.zeros((B, 5, N), jnp.float32)
    z2 = jnp.zeros((B, 5, S), jnp.float32)
    x1p = jnp.concatenate([xyz1, z1], axis=1)
    x2p = jnp.transpose(
        jnp.concatenate([xyz2, z2], axis=1), (0, 2, 1))

    gidx, wts = pl.pallas_call(
        _top3_body,
        grid=(B, NT),
        in_specs=[
            pl.BlockSpec((1, S, 8), lambda b, t: (b, 0, 0)),
            pl.BlockSpec((1, 8, TN), lambda b, t: (b, 0, t)),
            pl.BlockSpec((1, S, 1), lambda b, t: (b, 0, 0)),
            pl.BlockSpec((1, 1, TN), lambda b, t: (b, 0, t)),
        ],
        out_specs=[
            pl.BlockSpec((3, TN), lambda b, t: (0, b * (N // _TN) + t)),
            pl.BlockSpec((3, TN), lambda b, t: (0, b * (N // _TN) + t)),
        ],
        out_shape=[
            jax.ShapeDtypeStruct((3, Q), jnp.int32),
            jax.ShapeDtypeStruct((3, Q), jnp.float32),
        ],
        compiler_params=pltpu.CompilerParams(
            dimension_semantics=("parallel", "parallel")),
    )(x2p, x1p, n2, n1)

    p2l = jnp.reshape(jnp.transpose(points2, (0, 2, 1)), (B * S, D2))
    mesh = plsc.VectorSubcoreMesh(core_axis_name="c", subcore_axis_name="s")
    bag = functools.partial(
        pl.kernel,
        mesh=mesh,
        out_type=jax.ShapeDtypeStruct((Q, D2), jnp.float32),
        scratch_types=[
            pltpu.VMEM((Q // (_NC * _NS),), jnp.int32),
            pltpu.VMEM((Q // (_NC * _NS),), jnp.int32),
            pltpu.VMEM((Q // (_NC * _NS),), jnp.int32),
            pltpu.VMEM((Q // (_NC * _NS),), jnp.float32),
            pltpu.VMEM((Q // (_NC * _NS),), jnp.float32),
            pltpu.VMEM((Q // (_NC * _NS),), jnp.float32),
            pltpu.VMEM((_C, D2), jnp.float32),
            pltpu.VMEM((_C, D2), jnp.float32),
            pltpu.VMEM((_C, D2), jnp.float32),
            pltpu.VMEM((_C, D2), jnp.float32),
            pltpu.VMEM((_C, D2), jnp.float32),
            pltpu.VMEM((_C, D2), jnp.float32),
            pltpu.VMEM((_C, D2), jnp.float32),
            pltpu.SemaphoreType.DMA,
            pltpu.SemaphoreType.DMA,
        ],
    )(_bag_body)
    interp = bag(p2l, jnp.reshape(gidx, (3 * Q,)),
                 jnp.reshape(wts, (3 * Q,)))

    W1a = W1[:, :D1]
    W1b = W1[:, D1:]
    b1c = b1[:, None]
    b2c = b2[:, None]
    out = pl.pallas_call(
        _mlp_body,
        grid=(B, NT),
        in_specs=[
            pl.BlockSpec((TN, D2), lambda b, t: (b * (N // _TN) + t, 0)),
            pl.BlockSpec((1, D1, TN), lambda b, t: (b, 0, t)),
            pl.BlockSpec((H, D1), lambda b, t: (0, 0)),
            pl.BlockSpec((H, D2), lambda b, t: (0, 0)),
            pl.BlockSpec((O, H), lambda b, t: (0, 0)),
            pl.BlockSpec((H, 1), lambda b, t: (0, 0)),
            pl.BlockSpec((O, 1), lambda b, t: (0, 0)),
        ],
        out_specs=pl.BlockSpec((1, O, TN), lambda b, t: (b, 0, t)),
        out_shape=jax.ShapeDtypeStruct((B, O, N), jnp.float32),
        compiler_params=pltpu.CompilerParams(
            dimension_semantics=("parallel", "parallel")),
    )(interp, points1, W1a, W1b, W2, b1c, b2c)
    return out


# speedup vs baseline: 1.2627x; 1.1420x over previous
import functools

import jax
import jax.numpy as jnp
from jax import lax
from jax.experimental import pallas as pl
from jax.experimental.pallas import tpu as pltpu
from jax.experimental.pallas import tpu_sc as plsc

_TN = 512
_C = 32
_NSL = 2
_NC = 2
_NS = 16


def _top3_body(x2p_ref, x1p_ref, n2_ref, n1_ref, gidx_ref, wts_ref):
    x2p = x2p_ref[0]
    x1p = x1p_ref[0]
    S = x2p.shape[0]
    b = pl.program_id(0)

    mm = jax.lax.dot_general(
        x2p, x1p, (((1,), (0,)), ((), ())),
        preferred_element_type=jnp.float32)
    e0 = n2_ref[0] - 2.0 * mm
    iota = jax.lax.broadcasted_iota(jnp.int32, e0.shape, 0)

    e = e0
    ms = []
    for _ in range(3):
        m = jnp.min(e, axis=0, keepdims=True)
        ms.append(m)
        e = jnp.where(e == m, jnp.inf, e)

    rs = []
    for k in range(3):
        idx = jnp.min(jnp.where(e0 == ms[k], iota, S), axis=0,
                      keepdims=True)
        gidx_ref[pl.ds(k, 1), :] = idx + b * S
        rs.append(1.0 / (ms[k] + n1_ref[0] + 1e-8))
    norm = rs[0] + rs[1] + rs[2]
    for k in range(3):
        wts_ref[pl.ds(k, 1), :] = rs[k] / norm


def _bag_body(p2l_hbm, gidx_hbm, wts_hbm, interp_hbm,
              i0_v, i1_v, i2_v, w0_v, w1_v, w2_v,
              ra0, ra1, ra2, rb0, rb1, rb2, out_v, sema, semb):
    C = _C
    D2 = p2l_hbm.shape[1]
    Q = gidx_hbm.shape[0] // 3
    QW = Q // (_NC * _NS)
    NCH = QW // C
    wid = lax.axis_index("s") * _NC + lax.axis_index("c")
    span = wid * QW

    pltpu.sync_copy(gidx_hbm.at[pl.ds(span, QW)], i0_v)
    pltpu.sync_copy(gidx_hbm.at[pl.ds(Q + span, QW)], i1_v)
    pltpu.sync_copy(gidx_hbm.at[pl.ds(2 * Q + span, QW)], i2_v)
    pltpu.sync_copy(wts_hbm.at[pl.ds(span, QW)], w0_v)
    pltpu.sync_copy(wts_hbm.at[pl.ds(Q + span, QW)], w1_v)
    pltpu.sync_copy(wts_hbm.at[pl.ds(2 * Q + span, QW)], w2_v)

    def issue(c, bufs, sem):
        off = c * C
        r0, r1, r2 = bufs
        pltpu.async_copy(p2l_hbm.at[i0_v.at[pl.ds(off, C)]], r0, sem)
        pltpu.async_copy(p2l_hbm.at[i1_v.at[pl.ds(off, C)]], r1, sem)
        pltpu.async_copy(p2l_hbm.at[i2_v.at[pl.ds(off, C)]], r2, sem)

    def drain(c, bufs, sem):
        off = c * C
        r0, r1, r2 = bufs
        pltpu.make_async_copy(p2l_hbm.at[i0_v.at[pl.ds(off, C)]], r0, sem).wait()
        pltpu.make_async_copy(p2l_hbm.at[i1_v.at[pl.ds(off, C)]], r1, sem).wait()
        pltpu.make_async_copy(p2l_hbm.at[i2_v.at[pl.ds(off, C)]], r2, sem).wait()

    def compute(c, bufs):
        r0_v, r1_v, r2_v = bufs

        def group(g, carry2):
            base = g * 16
            w0v = w0_v[pl.ds(c * C + base, 16)]
            w1v = w1_v[pl.ds(c * C + base, 16)]
            w2v = w2_v[pl.ds(c * C + base, 16)]
            for ii in range(16):
                w0, w1, w2 = w0v[ii], w1v[ii], w2v[ii]
                i = base + ii
                for j in range(D2 // 16):
                    sl = pl.ds(j * 16, 16)
                    out_v[i, sl] = (w0 * r0_v[i, sl] + w1 * r1_v[i, sl]
                                    + w2 * r2_v[i, sl])
            return carry2

        lax.fori_loop(0, C // 16, group, 0)
        pltpu.sync_copy(out_v, interp_hbm.at[pl.ds(span + c * C, C)])

    bufs_a = (ra0, ra1, ra2)
    bufs_b = (rb0, rb1, rb2)
    issue(0, bufs_a, sema)

    def pair(p, carry):
        c0 = 2 * p
        c1 = c0 + 1
        issue(c1, bufs_b, semb)
        drain(c0, bufs_a, sema)
        compute(c0, bufs_a)

        @pl.when(c0 + 2 < NCH)
        def _():
            issue(c0 + 2, bufs_a, sema)

        drain(c1, bufs_b, semb)
        compute(c1, bufs_b)
        return carry

    lax.fori_loop(0, NCH // 2, pair, 0)


def _mlp_body(interp_ref, p1_ref, w1a_ref, w1b_ref, w2_ref, b1_ref, b2_ref,
              out_ref):
    h = (jnp.dot(w1a_ref[...], p1_ref[0],
                 preferred_element_type=jnp.float32)
         + jax.lax.dot_general(
             w1b_ref[...], interp_ref[...], (((1,), (1,)), ((), ())),
             preferred_element_type=jnp.float32)
         + b1_ref[...])
    h = jnp.maximum(h, 0.0)
    o = jnp.dot(w2_ref[...], h,
                preferred_element_type=jnp.float32) + b2_ref[...]
    out_ref[0] = jnp.maximum(o, 0.0)


def kernel(xyz1, xyz2, points1, points2, W1, b1, W2, b2):
    B, _, N = xyz1.shape
    S = xyz2.shape[2]
    D1 = points1.shape[1]
    D2 = points2.shape[1]
    H = W1.shape[0]
    O = W2.shape[0]
    TN = _TN
    NT = N // TN
    Q = B * N

    n1 = jnp.sum(xyz1 * xyz1, axis=1, keepdims=True)
    n2 = jnp.transpose(jnp.sum(xyz2 * xyz2, axis=1, keepdims=True),
                       (0, 2, 1))
    z1 = jnp.zeros((B, 5, N), jnp.float32)
    z2 = jnp.zeros((B, 5, S), jnp.float32)
    x1p = jnp.concatenate([xyz1, z1], axis=1)
    x2p = jnp.transpose(
        jnp.concatenate([xyz2, z2], axis=1), (0, 2, 1))

    p2l = jnp.reshape(jnp.transpose(points2, (0, 2, 1)), (B * S, D2))
    W1a = W1[:, :D1]
    W1b = W1[:, D1:]
    b1c = b1[:, None]
    b2c = b2[:, None]
    mesh = plsc.VectorSubcoreMesh(core_axis_name="c", subcore_axis_name="s")

    NSL = _NSL
    B2 = B // NSL
    Q2 = B2 * N
    outs = []
    for sl in range(NSL):
        bsl = slice(sl * B2, (sl + 1) * B2)
        gidx, wts = pl.pallas_call(
            _top3_body,
            grid=(B2, NT),
            in_specs=[
                pl.BlockSpec((1, S, 8), lambda b, t: (b, 0, 0)),
                pl.BlockSpec((1, 8, TN), lambda b, t: (b, 0, t)),
                pl.BlockSpec((1, S, 1), lambda b, t: (b, 0, 0)),
                pl.BlockSpec((1, 1, TN), lambda b, t: (b, 0, t)),
            ],
            out_specs=[
                pl.BlockSpec((3, TN), lambda b, t: (0, b * (N // _TN) + t)),
                pl.BlockSpec((3, TN), lambda b, t: (0, b * (N // _TN) + t)),
            ],
            out_shape=[
                jax.ShapeDtypeStruct((3, Q2), jnp.int32),
                jax.ShapeDtypeStruct((3, Q2), jnp.float32),
            ],
            compiler_params=pltpu.CompilerParams(
                dimension_semantics=("parallel", "parallel")),
        )(x2p[bsl], x1p[bsl], n2[bsl], n1[bsl])

        bag = functools.partial(
            pl.kernel,
            mesh=mesh,
            out_type=jax.ShapeDtypeStruct((Q2, D2), jnp.float32),
            scratch_types=[
                pltpu.VMEM((Q2 // (_NC * _NS),), jnp.int32),
                pltpu.VMEM((Q2 // (_NC * _NS),), jnp.int32),
                pltpu.VMEM((Q2 // (_NC * _NS),), jnp.int32),
                pltpu.VMEM((Q2 // (_NC * _NS),), jnp.float32),
                pltpu.VMEM((Q2 // (_NC * _NS),), jnp.float32),
                pltpu.VMEM((Q2 // (_NC * _NS),), jnp.float32),
                pltpu.VMEM((_C, D2), jnp.float32),
                pltpu.VMEM((_C, D2), jnp.float32),
                pltpu.VMEM((_C, D2), jnp.float32),
                pltpu.VMEM((_C, D2), jnp.float32),
                pltpu.VMEM((_C, D2), jnp.float32),
                pltpu.VMEM((_C, D2), jnp.float32),
                pltpu.VMEM((_C, D2), jnp.float32),
                pltpu.SemaphoreType.DMA,
                pltpu.SemaphoreType.DMA,
            ],
        )(_bag_body)
        interp = bag(p2l[sl * B2 * S:(sl + 1) * B2 * S],
                     jnp.reshape(gidx, (3 * Q2,)),
                     jnp.reshape(wts, (3 * Q2,)))

        out_s = pl.pallas_call(
            _mlp_body,
            grid=(B2, NT),
            in_specs=[
                pl.BlockSpec((TN, D2), lambda b, t: (b * (N // _TN) + t, 0)),
                pl.BlockSpec((1, D1, TN), lambda b, t: (b, 0, t)),
                pl.BlockSpec((H, D1), lambda b, t: (0, 0)),
                pl.BlockSpec((H, D2), lambda b, t: (0, 0)),
                pl.BlockSpec((O, H), lambda b, t: (0, 0)),
                pl.BlockSpec((H, 1), lambda b, t: (0, 0)),
                pl.BlockSpec((O, 1), lambda b, t: (0, 0)),
            ],
            out_specs=pl.BlockSpec((1, O, TN), lambda b, t: (b, 0, t)),
            out_shape=jax.ShapeDtypeStruct((B2, O, N), jnp.float32),
            compiler_params=pltpu.CompilerParams(
                dimension_semantics=("parallel", "parallel")),
        )(interp, points1[bsl], W1a, W1b, W2, b1c, b2c)
        outs.append(out_s)
    return jnp.concatenate(outs, axis=0)


# speedup vs baseline: 1.3670x; 1.0826x over previous
import functools

import jax
import jax.numpy as jnp
from jax import lax
from jax.experimental import pallas as pl
from jax.experimental.pallas import tpu as pltpu
from jax.experimental.pallas import tpu_sc as plsc

_TN = 512
_C = 32
_NSL = 4
_NC = 2
_NS = 16


def _top3_body(x2p_ref, x1p_ref, n2_ref, n1_ref, gidx_ref, wts_ref):
    x2p = x2p_ref[0]
    x1p = x1p_ref[0]
    S = x2p.shape[0]
    b = pl.program_id(0)

    mm = jax.lax.dot_general(
        x2p, x1p, (((1,), (0,)), ((), ())),
        preferred_element_type=jnp.float32)
    e0 = n2_ref[0] - 2.0 * mm
    iota = jax.lax.broadcasted_iota(jnp.int32, e0.shape, 0)

    e = e0
    ms = []
    for _ in range(3):
        m = jnp.min(e, axis=0, keepdims=True)
        ms.append(m)
        e = jnp.where(e == m, jnp.inf, e)

    rs = []
    for k in range(3):
        idx = jnp.min(jnp.where(e0 == ms[k], iota, S), axis=0,
                      keepdims=True)
        gidx_ref[pl.ds(k, 1), :] = idx + b * S
        rs.append(1.0 / (ms[k] + n1_ref[0] + 1e-8))
    norm = rs[0] + rs[1] + rs[2]
    for k in range(3):
        wts_ref[pl.ds(k, 1), :] = rs[k] / norm


def _bag_body(p2l_hbm, gidx_hbm, wts_hbm, interp_hbm,
              i0_v, i1_v, i2_v, w0_v, w1_v, w2_v,
              ra0, ra1, ra2, rb0, rb1, rb2, out_v, sema, semb):
    C = _C
    D2 = p2l_hbm.shape[1]
    Q = gidx_hbm.shape[0] // 3
    QW = Q // (_NC * _NS)
    NCH = QW // C
    wid = lax.axis_index("s") * _NC + lax.axis_index("c")
    span = wid * QW

    pltpu.sync_copy(gidx_hbm.at[pl.ds(span, QW)], i0_v)
    pltpu.sync_copy(gidx_hbm.at[pl.ds(Q + span, QW)], i1_v)
    pltpu.sync_copy(gidx_hbm.at[pl.ds(2 * Q + span, QW)], i2_v)
    pltpu.sync_copy(wts_hbm.at[pl.ds(span, QW)], w0_v)
    pltpu.sync_copy(wts_hbm.at[pl.ds(Q + span, QW)], w1_v)
    pltpu.sync_copy(wts_hbm.at[pl.ds(2 * Q + span, QW)], w2_v)

    def issue(c, bufs, sem):
        off = c * C
        r0, r1, r2 = bufs
        pltpu.async_copy(p2l_hbm.at[i0_v.at[pl.ds(off, C)]], r0, sem)
        pltpu.async_copy(p2l_hbm.at[i1_v.at[pl.ds(off, C)]], r1, sem)
        pltpu.async_copy(p2l_hbm.at[i2_v.at[pl.ds(off, C)]], r2, sem)

    def drain(c, bufs, sem):
        off = c * C
        r0, r1, r2 = bufs
        pltpu.make_async_copy(p2l_hbm.at[i0_v.at[pl.ds(off, C)]], r0, sem).wait()
        pltpu.make_async_copy(p2l_hbm.at[i1_v.at[pl.ds(off, C)]], r1, sem).wait()
        pltpu.make_async_copy(p2l_hbm.at[i2_v.at[pl.ds(off, C)]], r2, sem).wait()

    def compute(c, bufs):
        r0_v, r1_v, r2_v = bufs

        def group(g, carry2):
            base = g * 16
            w0v = w0_v[pl.ds(c * C + base, 16)]
            w1v = w1_v[pl.ds(c * C + base, 16)]
            w2v = w2_v[pl.ds(c * C + base, 16)]
            for ii in range(16):
                w0, w1, w2 = w0v[ii], w1v[ii], w2v[ii]
                i = base + ii
                for j in range(D2 // 16):
                    sl = pl.ds(j * 16, 16)
                    out_v[i, sl] = (w0 * r0_v[i, sl] + w1 * r1_v[i, sl]
                                    + w2 * r2_v[i, sl])
            return carry2

        lax.fori_loop(0, C // 16, group, 0)
        pltpu.sync_copy(out_v, interp_hbm.at[pl.ds(span + c * C, C)])

    bufs_a = (ra0, ra1, ra2)
    bufs_b = (rb0, rb1, rb2)
    issue(0, bufs_a, sema)

    def pair(p, carry):
        c0 = 2 * p
        c1 = c0 + 1
        issue(c1, bufs_b, semb)
        drain(c0, bufs_a, sema)
        compute(c0, bufs_a)

        @pl.when(c0 + 2 < NCH)
        def _():
            issue(c0 + 2, bufs_a, sema)

        drain(c1, bufs_b, semb)
        compute(c1, bufs_b)
        return carry

    lax.fori_loop(0, NCH // 2, pair, 0)


def _mlp_body(interp_ref, p1_ref, w1a_ref, w1b_ref, w2_ref, b1_ref, b2_ref,
              out_ref):
    h = (jnp.dot(w1a_ref[...], p1_ref[0],
                 preferred_element_type=jnp.float32)
         + jax.lax.dot_general(
             w1b_ref[...], interp_ref[...], (((1,), (1,)), ((), ())),
             preferred_element_type=jnp.float32)
         + b1_ref[...])
    h = jnp.maximum(h, 0.0)
    o = jnp.dot(w2_ref[...], h,
                preferred_element_type=jnp.float32) + b2_ref[...]
    out_ref[0] = jnp.maximum(o, 0.0)


def kernel(xyz1, xyz2, points1, points2, W1, b1, W2, b2):
    B, _, N = xyz1.shape
    S = xyz2.shape[2]
    D1 = points1.shape[1]
    D2 = points2.shape[1]
    H = W1.shape[0]
    O = W2.shape[0]
    TN = _TN
    NT = N // TN
    Q = B * N

    n1 = jnp.sum(xyz1 * xyz1, axis=1, keepdims=True)
    n2 = jnp.transpose(jnp.sum(xyz2 * xyz2, axis=1, keepdims=True),
                       (0, 2, 1))
    z1 = jnp.zeros((B, 5, N), jnp.float32)
    z2 = jnp.zeros((B, 5, S), jnp.float32)
    x1p = jnp.concatenate([xyz1, z1], axis=1)
    x2p = jnp.transpose(
        jnp.concatenate([xyz2, z2], axis=1), (0, 2, 1))

    p2l = jnp.reshape(jnp.transpose(points2, (0, 2, 1)), (B * S, D2))
    W1a = W1[:, :D1]
    W1b = W1[:, D1:]
    b1c = b1[:, None]
    b2c = b2[:, None]
    mesh = plsc.VectorSubcoreMesh(core_axis_name="c", subcore_axis_name="s")

    NSL = _NSL
    B2 = B // NSL
    Q2 = B2 * N
    outs = []
    for sl in range(NSL):
        bsl = slice(sl * B2, (sl + 1) * B2)
        gidx, wts = pl.pallas_call(
            _top3_body,
            grid=(B2, NT),
            in_specs=[
                pl.BlockSpec((1, S, 8), lambda b, t: (b, 0, 0)),
                pl.BlockSpec((1, 8, TN), lambda b, t: (b, 0, t)),
                pl.BlockSpec((1, S, 1), lambda b, t: (b, 0, 0)),
                pl.BlockSpec((1, 1, TN), lambda b, t: (b, 0, t)),
            ],
            out_specs=[
                pl.BlockSpec((3, TN), lambda b, t: (0, b * (N // _TN) + t)),
                pl.BlockSpec((3, TN), lambda b, t: (0, b * (N // _TN) + t)),
            ],
            out_shape=[
                jax.ShapeDtypeStruct((3, Q2), jnp.int32),
                jax.ShapeDtypeStruct((3, Q2), jnp.float32),
            ],
            compiler_params=pltpu.CompilerParams(
                dimension_semantics=("parallel", "parallel")),
        )(x2p[bsl], x1p[bsl], n2[bsl], n1[bsl])

        bag = functools.partial(
            pl.kernel,
            mesh=mesh,
            out_type=jax.ShapeDtypeStruct((Q2, D2), jnp.float32),
            scratch_types=[
                pltpu.VMEM((Q2 // (_NC * _NS),), jnp.int32),
                pltpu.VMEM((Q2 // (_NC * _NS),), jnp.int32),
                pltpu.VMEM((Q2 // (_NC * _NS),), jnp.int32),
                pltpu.VMEM((Q2 // (_NC * _NS),), jnp.float32),
                pltpu.VMEM((Q2 // (_NC * _NS),), jnp.float32),
                pltpu.VMEM((Q2 // (_NC * _NS),), jnp.float32),
                pltpu.VMEM((_C, D2), jnp.float32),
                pltpu.VMEM((_C, D2), jnp.float32),
                pltpu.VMEM((_C, D2), jnp.float32),
                pltpu.VMEM((_C, D2), jnp.float32),
                pltpu.VMEM((_C, D2), jnp.float32),
                pltpu.VMEM((_C, D2), jnp.float32),
                pltpu.VMEM((_C, D2), jnp.float32),
                pltpu.SemaphoreType.DMA,
                pltpu.SemaphoreType.DMA,
            ],
        )(_bag_body)
        interp = bag(p2l[sl * B2 * S:(sl + 1) * B2 * S],
                     jnp.reshape(gidx, (3 * Q2,)),
                     jnp.reshape(wts, (3 * Q2,)))

        out_s = pl.pallas_call(
            _mlp_body,
            grid=(B2, NT),
            in_specs=[
                pl.BlockSpec((TN, D2), lambda b, t: (b * (N // _TN) + t, 0)),
                pl.BlockSpec((1, D1, TN), lambda b, t: (b, 0, t)),
                pl.BlockSpec((H, D1), lambda b, t: (0, 0)),
                pl.BlockSpec((H, D2), lambda b, t: (0, 0)),
                pl.BlockSpec((O, H), lambda b, t: (0, 0)),
                pl.BlockSpec((H, 1), lambda b, t: (0, 0)),
                pl.BlockSpec((O, 1), lambda b, t: (0, 0)),
            ],
            out_specs=pl.BlockSpec((1, O, TN), lambda b, t: (b, 0, t)),
            out_shape=jax.ShapeDtypeStruct((B2, O, N), jnp.float32),
            compiler_params=pltpu.CompilerParams(
                dimension_semantics=("parallel", "parallel")),
        )(interp, points1[bsl], W1a, W1b, W2, b1c, b2c)
        outs.append(out_s)
    return jnp.concatenate(outs, axis=0)


# speedup vs baseline: 1.3690x; 1.0015x over previous
import functools

import jax
import jax.numpy as jnp
from jax import lax
from jax.experimental import pallas as pl
from jax.experimental.pallas import tpu as pltpu
from jax.experimental.pallas import tpu_sc as plsc

_TN = 512
_C = 32
_NSL = 4
_NC = 2
_NS = 16


def _top3_body(x2p_ref, x1p_ref, n2_ref, n1_ref, gidx_ref, wts_ref):
    x2p = x2p_ref[0]
    x1p = x1p_ref[0]
    S = x2p.shape[0]
    b = pl.program_id(0)

    mm = jax.lax.dot_general(
        x2p, x1p, (((1,), (0,)), ((), ())),
        preferred_element_type=jnp.float32)
    d0 = n2_ref[0] + n1_ref[0] - 2.0 * mm
    iota = jax.lax.broadcasted_iota(jnp.int32, d0.shape, 0)

    d = d0
    ms = []
    for _ in range(3):
        m = jnp.min(d, axis=0, keepdims=True)
        ms.append(m)
        d = jnp.where(d == m, jnp.inf, d)

    rs = []
    for k in range(3):
        idx = jnp.min(jnp.where(d0 == ms[k], iota, S), axis=0,
                      keepdims=True)
        gidx_ref[pl.ds(k, 1), :] = idx + b * S
        rs.append(1.0 / (ms[k] + 1e-8))
    norm = rs[0] + rs[1] + rs[2]
    for k in range(3):
        wts_ref[pl.ds(k, 1), :] = rs[k] / norm


def _bag_body(p2l_hbm, gidx_hbm, wts_hbm, interp_hbm,
              i0_v, i1_v, i2_v, w0_v, w1_v, w2_v,
              ra0, ra1, ra2, rb0, rb1, rb2, out_v, sema, semb):
    C = _C
    D2 = p2l_hbm.shape[1]
    Q = gidx_hbm.shape[0] // 3
    QW = Q // (_NC * _NS)
    NCH = QW // C
    wid = lax.axis_index("s") * _NC + lax.axis_index("c")
    span = wid * QW

    pltpu.sync_copy(gidx_hbm.at[pl.ds(span, QW)], i0_v)
    pltpu.sync_copy(gidx_hbm.at[pl.ds(Q + span, QW)], i1_v)
    pltpu.sync_copy(gidx_hbm.at[pl.ds(2 * Q + span, QW)], i2_v)
    pltpu.sync_copy(wts_hbm.at[pl.ds(span, QW)], w0_v)
    pltpu.sync_copy(wts_hbm.at[pl.ds(Q + span, QW)], w1_v)
    pltpu.sync_copy(wts_hbm.at[pl.ds(2 * Q + span, QW)], w2_v)

    def issue(c, bufs, sem):
        off = c * C
        r0, r1, r2 = bufs
        pltpu.async_copy(p2l_hbm.at[i0_v.at[pl.ds(off, C)]], r0, sem)
        pltpu.async_copy(p2l_hbm.at[i1_v.at[pl.ds(off, C)]], r1, sem)
        pltpu.async_copy(p2l_hbm.at[i2_v.at[pl.ds(off, C)]], r2, sem)

    def drain(c, bufs, sem):
        off = c * C
        r0, r1, r2 = bufs
        pltpu.make_async_copy(p2l_hbm.at[i0_v.at[pl.ds(off, C)]], r0, sem).wait()
        pltpu.make_async_copy(p2l_hbm.at[i1_v.at[pl.ds(off, C)]], r1, sem).wait()
        pltpu.make_async_copy(p2l_hbm.at[i2_v.at[pl.ds(off, C)]], r2, sem).wait()

    def compute(c, bufs):
        r0_v, r1_v, r2_v = bufs

        def group(g, carry2):
            base = g * 16
            w0v = w0_v[pl.ds(c * C + base, 16)]
            w1v = w1_v[pl.ds(c * C + base, 16)]
            w2v = w2_v[pl.ds(c * C + base, 16)]
            for ii in range(16):
                w0, w1, w2 = w0v[ii], w1v[ii], w2v[ii]
                i = base + ii
                for j in range(D2 // 16):
                    sl = pl.ds(j * 16, 16)
                    out_v[i, sl] = (w0 * r0_v[i, sl] + w1 * r1_v[i, sl]
                                    + w2 * r2_v[i, sl])
            return carry2

        lax.fori_loop(0, C // 16, group, 0)
        pltpu.sync_copy(out_v, interp_hbm.at[pl.ds(span + c * C, C)])

    bufs_a = (ra0, ra1, ra2)
    bufs_b = (rb0, rb1, rb2)
    issue(0, bufs_a, sema)

    def pair(p, carry):
        c0 = 2 * p
        c1 = c0 + 1
        issue(c1, bufs_b, semb)
        drain(c0, bufs_a, sema)
        compute(c0, bufs_a)

        @pl.when(c0 + 2 < NCH)
        def _():
            issue(c0 + 2, bufs_a, sema)

        drain(c1, bufs_b, semb)
        compute(c1, bufs_b)
        return carry

    lax.fori_loop(0, NCH // 2, pair, 0)


def _mlp_body(interp_ref, p1_ref, w1a_ref, w1b_ref, w2_ref, b1_ref, b2_ref,
              out_ref):
    h = (jnp.dot(w1a_ref[...], p1_ref[0],
                 preferred_element_type=jnp.float32)
         + jax.lax.dot_general(
             w1b_ref[...], interp_ref[...], (((1,), (1,)), ((), ())),
             preferred_element_type=jnp.float32)
         + b1_ref[...])
    h = jnp.maximum(h, 0.0)
    o = jnp.dot(w2_ref[...], h,
                preferred_element_type=jnp.float32) + b2_ref[...]
    out_ref[0] = jnp.maximum(o, 0.0)


def kernel(xyz1, xyz2, points1, points2, W1, b1, W2, b2):
    B, _, N = xyz1.shape
    S = xyz2.shape[2]
    D1 = points1.shape[1]
    D2 = points2.shape[1]
    H = W1.shape[0]
    O = W2.shape[0]
    TN = _TN
    NT = N // TN
    Q = B * N

    n1 = jnp.sum(xyz1 * xyz1, axis=1, keepdims=True)
    n2 = jnp.transpose(jnp.sum(xyz2 * xyz2, axis=1, keepdims=True),
                       (0, 2, 1))
    z1 = jnp.zeros((B, 5, N), jnp.float32)
    z2 = jnp.zeros((B, 5, S), jnp.float32)
    x1p = jnp.concatenate([xyz1, z1], axis=1)
    x2p = jnp.transpose(
        jnp.concatenate([xyz2, z2], axis=1), (0, 2, 1))

    p2l = jnp.reshape(jnp.transpose(points2, (0, 2, 1)), (B * S, D2))
    W1a = W1[:, :D1]
    W1b = W1[:, D1:]
    b1c = b1[:, None]
    b2c = b2[:, None]
    mesh = plsc.VectorSubcoreMesh(core_axis_name="c", subcore_axis_name="s")

    NSL = _NSL
    B2 = B // NSL
    Q2 = B2 * N
    outs = []
    for sl in range(NSL):
        bsl = slice(sl * B2, (sl + 1) * B2)
        gidx, wts = pl.pallas_call(
            _top3_body,
            grid=(B2, NT),
            in_specs=[
                pl.BlockSpec((1, S, 8), lambda b, t: (b, 0, 0)),
                pl.BlockSpec((1, 8, TN), lambda b, t: (b, 0, t)),
                pl.BlockSpec((1, S, 1), lambda b, t: (b, 0, 0)),
                pl.BlockSpec((1, 1, TN), lambda b, t: (b, 0, t)),
            ],
            out_specs=[
                pl.BlockSpec((3, TN), lambda b, t: (0, b * (N // _TN) + t)),
                pl.BlockSpec((3, TN), lambda b, t: (0, b * (N // _TN) + t)),
            ],
            out_shape=[
                jax.ShapeDtypeStruct((3, Q2), jnp.int32),
                jax.ShapeDtypeStruct((3, Q2), jnp.float32),
            ],
            compiler_params=pltpu.CompilerParams(
                dimension_semantics=("parallel", "parallel")),
        )(x2p[bsl], x1p[bsl], n2[bsl], n1[bsl])

        bag = functools.partial(
            pl.kernel,
            mesh=mesh,
            out_type=jax.ShapeDtypeStruct((Q2, D2), jnp.float32),
            scratch_types=[
                pltpu.VMEM((Q2 // (_NC * _NS),), jnp.int32),
                pltpu.VMEM((Q2 // (_NC * _NS),), jnp.int32),
                pltpu.VMEM((Q2 // (_NC * _NS),), jnp.int32),
                pltpu.VMEM((Q2 // (_NC * _NS),), jnp.float32),
                pltpu.VMEM((Q2 // (_NC * _NS),), jnp.float32),
                pltpu.VMEM((Q2 // (_NC * _NS),), jnp.float32),
                pltpu.VMEM((_C, D2), jnp.float32),
                pltpu.VMEM((_C, D2), jnp.float32),
                pltpu.VMEM((_C, D2), jnp.float32),
                pltpu.VMEM((_C, D2), jnp.float32),
                pltpu.VMEM((_C, D2), jnp.float32),
                pltpu.VMEM((_C, D2), jnp.float32),
                pltpu.VMEM((_C, D2), jnp.float32),
                pltpu.SemaphoreType.DMA,
                pltpu.SemaphoreType.DMA,
            ],
        )(_bag_body)
        interp = bag(p2l[sl * B2 * S:(sl + 1) * B2 * S],
                     jnp.reshape(gidx, (3 * Q2,)),
                     jnp.reshape(wts, (3 * Q2,)))

        out_s = pl.pallas_call(
            _mlp_body,
            grid=(B2, NT),
            in_specs=[
                pl.BlockSpec((TN, D2), lambda b, t: (b * (N // _TN) + t, 0)),
                pl.BlockSpec((1, D1, TN), lambda b, t: (b, 0, t)),
                pl.BlockSpec((H, D1), lambda b, t: (0, 0)),
                pl.BlockSpec((H, D2), lambda b, t: (0, 0)),
                pl.BlockSpec((O, H), lambda b, t: (0, 0)),
                pl.BlockSpec((H, 1), lambda b, t: (0, 0)),
                pl.BlockSpec((O, 1), lambda b, t: (0, 0)),
            ],
            out_specs=pl.BlockSpec((1, O, TN), lambda b, t: (b, 0, t)),
            out_shape=jax.ShapeDtypeStruct((B2, O, N), jnp.float32),
            compiler_params=pltpu.CompilerParams(
                dimension_semantics=("parallel", "parallel")),
        )(interp, points1[bsl], W1a, W1b, W2, b1c, b2c)
        outs.append(out_s)
    return jnp.concatenate(outs, axis=0)
